# baseline (speedup 1.0000x reference)
import functools

import jax
import jax.numpy as jnp
from jax import lax
from jax.experimental import pallas as pl
from jax.experimental.pallas import tpu as pltpu
from jax.experimental.pallas import tpu_sc as plsc

_NC = 2
_NS = 16
_NW = _NC * _NS


def kernel(x, layer_id, tables):
    n_layers, vocab, d = tables.shape
    b, s = x.shape
    n = b * s
    b_per_w = n // _NW
    sub_per_row = s // b_per_w
    flat_tables = tables.reshape(n_layers * vocab, d)
    lid = jnp.asarray(layer_id, jnp.int32).reshape(1)

    mesh = plsc.VectorSubcoreMesh(core_axis_name="c", subcore_axis_name="s")

    @functools.partial(
        pl.kernel,
        mesh=mesh,
        out_type=jax.ShapeDtypeStruct((n, d), tables.dtype),
        scratch_types=[
            pltpu.VMEM((b_per_w,), jnp.int32),
            pltpu.VMEM((16,), jnp.int32),
            pltpu.VMEM((b_per_w, d), jnp.float32),
            pltpu.SemaphoreType.DMA,
            pltpu.SemaphoreType.DMA,
            pltpu.SemaphoreType.DMA,
            pltpu.SemaphoreType.DMA,
        ],
    )
    def gather_kernel(table_hbm, x_hbm, lid_hbm, out_hbm,
                      idx_v, lid_v, rows_v, sem_a, sem_b, sem_c, sem_d):
        wid = lax.axis_index("s") * _NC + lax.axis_index("c")
        row = wid // sub_per_row
        col = (wid % sub_per_row) * b_per_w
        half = b_per_w // 2
        obase = wid * b_per_w
        c_lid = pltpu.async_copy(lid_hbm, lid_v.at[pl.ds(0, 1)], sem_a)
        c_idx = pltpu.async_copy(x_hbm.at[row].at[pl.ds(col, b_per_w)],
                                 idx_v, sem_b)
        c_lid.wait()
        c_idx.wait()
        base = lid_v[...][0] * vocab
        view = table_hbm.at[pl.ds(base, vocab)]
        q = b_per_w // 4
        sems = (sem_a, sem_b, sem_c, sem_d)
        gathers = []
        for k in range(4):
            gathers.append(pltpu.async_copy(
                view.at[idx_v.at[pl.ds(k * q, q)]],
                rows_v.at[pl.ds(k * q, q)], sems[k]))
        stores = []
        for k in range(4):
            gathers[k].wait()
            stores.append(pltpu.async_copy(
                rows_v.at[pl.ds(k * q, q)],
                out_hbm.at[pl.ds(obase + k * q, q)], sems[k]))
        for k in range(4):
            stores[k].wait()

    out = gather_kernel(flat_tables, x, lid)
    return out.reshape(b, s, d)


# speedup vs baseline: 1.0013x; 1.0013x over previous
import functools

import jax
import jax.numpy as jnp
from jax import lax
from jax.experimental import pallas as pl
from jax.experimental.pallas import tpu as pltpu
from jax.experimental.pallas import tpu_sc as plsc

_NC = 2
_NS = 16
_NW = _NC * _NS


def kernel(x, layer_id, tables):
    n_layers, vocab, d = tables.shape
    b, s = x.shape
    n = b * s
    b_per_w = n // _NW
    sub_per_row = s // b_per_w
    flat_tables = tables.reshape(n_layers * vocab, d)
    lid = jnp.asarray(layer_id, jnp.int32).reshape(1)

    mesh = plsc.VectorSubcoreMesh(core_axis_name="c", subcore_axis_name="s")

    @functools.partial(
        pl.kernel,
        mesh=mesh,
        out_type=jax.ShapeDtypeStruct((n, d), tables.dtype),
        scratch_types=[
            pltpu.VMEM((b_per_w,), jnp.int32),
            pltpu.VMEM((16,), jnp.int32),
            pltpu.VMEM((b_per_w, d), jnp.float32),
            pltpu.SemaphoreType.DMA,
            pltpu.SemaphoreType.DMA,
            pltpu.SemaphoreType.DMA,
            pltpu.SemaphoreType.DMA,
        ],
    )
    def gather_kernel(table_hbm, x_hbm, lid_hbm, out_hbm,
                      idx_v, lid_v, rows_v, sem_a, sem_b, sem_c, sem_d):
        wid = lax.axis_index("s") * _NC + lax.axis_index("c")
        row = wid // sub_per_row
        col = (wid % sub_per_row) * b_per_w
        half = b_per_w // 2
        obase = wid * b_per_w
        c_lid = pltpu.async_copy(lid_hbm, lid_v.at[pl.ds(0, 1)], sem_a)
        c_idx = pltpu.async_copy(x_hbm.at[row].at[pl.ds(col, b_per_w)],
                                 idx_v, sem_b)
        c_lid.wait()
        c_idx.wait()
        base = lid_v[...][0] * vocab
        view = table_hbm.at[pl.ds(base, vocab)]
        g0 = pltpu.async_copy(view.at[idx_v.at[pl.ds(0, half)]],
                              rows_v.at[pl.ds(0, half)], sem_a)
        g1 = pltpu.async_copy(view.at[idx_v.at[pl.ds(half, half)]],
                              rows_v.at[pl.ds(half, half)], sem_b)
        g0.wait()
        s0 = pltpu.async_copy(rows_v.at[pl.ds(0, half)],
                              out_hbm.at[pl.ds(obase, half)], sem_c)
        g1.wait()
        s1 = pltpu.async_copy(rows_v.at[pl.ds(half, half)],
                              out_hbm.at[pl.ds(obase + half, half)], sem_d)
        s0.wait()
        s1.wait()

    out = gather_kernel(flat_tables, x, lid)
    return out.reshape(b, s, d)
